# Initial kernel scaffold; baseline (speedup 1.0000x reference)
import jax
import jax.numpy as jnp
from jax.experimental import pallas as pl


def kernel(xyz, feat, d0_w0, d0_b0, d0_w1, d0_b1, d1_w0, d1_b0, d1_w1, d1_b1, d2_w0, d2_b0, d2_w1, d2_b1, u0_w0, u0_b0, u0_w1, u0_b1, u1_w0, u1_b0, u1_w1, u1_b1):
    raise NotImplementedError("write your pallas kernel here")


# speedup vs baseline: 3.2026x; 3.2026x over previous
import functools

import jax
import jax.numpy as jnp
from jax.experimental import pallas as pl
from jax.experimental.pallas import tpu as pltpu

def _mm(a, b, precision=jax.lax.Precision.DEFAULT):
    return jax.lax.dot_general(a, b, (((1,), (0,)), ((), ())),
                               precision=precision,
                               preferred_element_type=jnp.float32)


def _select_pool(q, s6_ref, sall_ref, d2_ref, nsample, r2, chunk, apply_mlp,
                 out_dim):
    bq = q.shape[0]
    n = sall_ref.shape[0]
    nchunks = n // chunk
    inf = jnp.float32(jnp.inf)

    qq = jnp.sum(q * q, axis=1, keepdims=True)

    vals_l, idxs_l = [], []
    for c in range(nchunks):
        sl = pl.ds(c * chunk, chunk)
        sx = s6_ref[:3, sl]
        ss = jnp.sum(sx * sx, axis=0, keepdims=True)
        qs = _mm(q, sx)
        d2c = (qq + ss) - 2.0 * qs
        d2_ref[:, sl] = d2c
        iota_c = jax.lax.broadcasted_iota(jnp.int32, (bq, chunk), 1) + c * chunk
        mc = jnp.min(d2c, axis=1, keepdims=True)
        ac = jnp.min(jnp.where(d2c == mc, iota_c, n), axis=1, keepdims=True)
        vals_l.append(mc)
        idxs_l.append(ac)
    vals = jnp.concatenate(vals_l, axis=1)
    idxs = jnp.concatenate(idxs_l, axis=1)

    def step(j, carry):
        acc, vals, idxs = carry
        m = jnp.min(vals, axis=1, keepdims=True)
        a = jnp.min(jnp.where(vals == m, idxs, n), axis=1,
                    keepdims=True)
        g = jnp.zeros((bq, sall_ref.shape[1]), jnp.float32)
        vals_n, idxs_n = [], []
        for c in range(nchunks):
            sl = pl.ds(c * chunk, chunk)
            d2c = d2_ref[:, sl]
            iota_c = (jax.lax.broadcasted_iota(jnp.int32, (bq, chunk), 1)
                      + c * chunk)
            sel = iota_c == a
            d2c = jnp.where(sel, inf, d2c)
            d2_ref[:, sl] = d2c
            g = g + _mm(sel.astype(jnp.float32), sall_ref[sl, :],
                        precision=jax.lax.Precision.HIGHEST)
            mc = jnp.min(d2c, axis=1, keepdims=True)
            ac = jnp.min(jnp.where(d2c == mc, iota_c, n), axis=1,
                         keepdims=True)
            vals_n.append(mc)
            idxs_n.append(ac)
        rel = g[:, :3] - q
        gg = jnp.concatenate([rel, g[:, 3:]], axis=1)
        h = apply_mlp(gg)
        upd = jnp.logical_or(m <= r2, j == 0)
        acc = jnp.where(upd, jnp.maximum(acc, h), acc)
        return (acc, jnp.concatenate(vals_n, axis=1),
                jnp.concatenate(idxs_n, axis=1))

    acc0 = jnp.full((bq, out_dim), -inf, jnp.float32)
    acc, _, _ = jax.lax.fori_loop(0, nsample, step, (acc0, vals, idxs))
    return acc


def _down_body(nsample, r2, chunk, q_ref, s6_ref, sall_ref, w1_ref, b1_ref,
               w2_ref, b2_ref, o_ref, d2_ref):
    w1 = w1_ref[...]
    b1 = b1_ref[...]
    w2 = w2_ref[...]
    b2 = b2_ref[...]

    def mlp(gg):
        h = jnp.maximum(_mm(gg, w1) + b1, 0.0)
        return jnp.maximum(_mm(h, w2) + b2, 0.0)

    o_ref[0] = _select_pool(q_ref[0], s6_ref.at[0], sall_ref.at[0], d2_ref,
                            nsample, r2, chunk, mlp, w2.shape[1])


def _up_body(nsample, r2, chunk, q_ref, s6_ref, sall_ref, fd_ref, w1_ref,
             b1_ref, w2_ref, b2_ref, o_ref, d2_ref):
    w1 = w1_ref[...]
    b1 = b1_ref[...]
    w2 = w2_ref[...]
    b2 = b2_ref[...]

    def mlp(gg):
        return jnp.maximum(_mm(gg, w1) + b1, 0.0)

    pooled = _select_pool(q_ref[0], s6_ref.at[0], sall_ref.at[0], d2_ref,
                          nsample, r2, chunk, mlp, w1.shape[1])
    hh = jnp.concatenate([pooled, fd_ref[0]], axis=1)
    o_ref[0] = jnp.maximum(_mm(hh, w2) + b2, 0.0)


def _stage(q_xyz, s_xyz, s_feat, w1, b1, w2, b2, nsample, radius, bq,
           q_feat=None):
    B, M, _ = q_xyz.shape
    _, N, C = s_feat.shape
    sall = jnp.concatenate([s_xyz, s_feat], axis=2)
    s6 = jnp.transpose(sall, (0, 2, 1))
    chunk = min(512, N)
    F2 = w2.shape[1]
    up = q_feat is not None
    body = functools.partial(_up_body if up else _down_body,
                             nsample, radius * radius, chunk)
    in_specs = [
        pl.BlockSpec((1, bq, 3), lambda b, i: (b, i, 0)),
        pl.BlockSpec((1, 3 + C, N), lambda b, i: (b, 0, 0)),
        pl.BlockSpec((1, N, 3 + C), lambda b, i: (b, 0, 0)),
    ]
    args = [q_xyz, s6, sall]
    if up:
        in_specs.append(
            pl.BlockSpec((1, bq, q_feat.shape[2]), lambda b, i: (b, i, 0)))
        args.append(q_feat)
    in_specs += [
        pl.BlockSpec(w1.shape, lambda b, i: (0, 0)),
        pl.BlockSpec((1, w1.shape[1]), lambda b, i: (0, 0)),
        pl.BlockSpec(w2.shape, lambda b, i: (0, 0)),
        pl.BlockSpec((1, w2.shape[1]), lambda b, i: (0, 0)),
    ]
    args += [w1, b1.reshape(1, -1), w2, b2.reshape(1, -1)]
    return pl.pallas_call(
        body,
        grid=(B, M // bq),
        in_specs=in_specs,
        out_specs=pl.BlockSpec((1, bq, F2), lambda b, i: (b, i, 0)),
        out_shape=jax.ShapeDtypeStruct((B, M, F2), jnp.float32),
        scratch_shapes=[pltpu.VMEM((bq, N), jnp.float32)],
    )(*args)


def kernel(xyz, feat, d0_w0, d0_b0, d0_w1, d0_b1, d1_w0, d1_b0, d1_w1,
           d1_b1, d2_w0, d2_b0, d2_w1, d2_b1, u0_w0, u0_b0, u0_w1, u0_b1,
           u1_w0, u1_b0, u1_w1, u1_b1):
    B = xyz.shape[0]
    x1 = xyz[:, ::4]
    x2 = x1[:, ::4]
    x3 = x2[:, ::4]

    f1 = _stage(x1, xyz, feat, d0_w0, d0_b0, d0_w1, d0_b1,
                nsample=32, radius=0.1, bq=256)
    f2 = _stage(x2, x1, f1, d1_w0, d1_b0, d1_w1, d1_b1,
                nsample=32, radius=0.2, bq=512)
    f3 = _stage(x3, x2, f2, d2_w0, d2_b0, d2_w1, d2_b1,
                nsample=32, radius=0.4, bq=128)
    u0 = _stage(x2, x3, f3, u0_w0, u0_b0, u0_w1, u0_b1,
                nsample=32, radius=0.4, bq=512, q_feat=f2)
    u1 = _stage(x1, x2, u0, u1_w0, u1_b0, u1_w1, u1_b1,
                nsample=32, radius=0.2, bq=512, q_feat=f1)

    idx = jnp.broadcast_to(
        (jnp.arange(2048, dtype=jnp.int32) * 4)[None, :], (B, 2048))
    return (u1, x1, idx)


# speedup vs baseline: 6.7816x; 2.1175x over previous
import functools

import jax
import jax.numpy as jnp
from jax.experimental import pallas as pl
from jax.experimental.pallas import tpu as pltpu


def _mm(a, b):
    return jax.lax.dot_general(a, b, (((1,), (0,)), ((), ())),
                               precision=jax.lax.Precision.DEFAULT,
                               preferred_element_type=jnp.float32)


def _select_pool(q, sx_ref, hi_ref, lo_ref, d2_ref, nsample, r2, chunk,
                 apply_mlp, out_dim):
    bq = q.shape[0]
    n = hi_ref.shape[0]
    nchunks = n // chunk
    wide = hi_ref.shape[1]
    inf = jnp.float32(jnp.inf)

    qq = jnp.sum(q * q, axis=1, keepdims=True)

    vals_l, idxs_l = [], []
    for c in range(nchunks):
        sl = pl.ds(c * chunk, chunk)
        sx = sx_ref[:, sl]
        ss = jnp.sum(sx * sx, axis=0, keepdims=True)
        qs = _mm(q, sx)
        d2c = (qq + ss) - 2.0 * qs
        d2_ref[:, sl] = d2c
        iota_c = (jax.lax.broadcasted_iota(jnp.int32, (bq, chunk), 1)
                  + c * chunk)
        mc = jnp.min(d2c, axis=1, keepdims=True)
        ac = jnp.min(jnp.where(d2c == mc, iota_c, n), axis=1, keepdims=True)
        vals_l.append(mc)
        idxs_l.append(ac)
    vals = jnp.concatenate(vals_l, axis=1)
    idxs = jnp.concatenate(idxs_l, axis=1)

    def step(j, carry):
        acc, vals, idxs = carry
        m = jnp.min(vals, axis=1, keepdims=True)
        a = jnp.min(jnp.where(vals == m, idxs, n), axis=1, keepdims=True)
        g = jnp.zeros((bq, wide), jnp.float32)
        vals_n, idxs_n = [], []
        for c in range(nchunks):
            sl = pl.ds(c * chunk, chunk)
            d2c = d2_ref[:, sl]
            iota_c = (jax.lax.broadcasted_iota(jnp.int32, (bq, chunk), 1)
                      + c * chunk)
            sel = iota_c == a
            d2c = jnp.where(sel, inf, d2c)
            d2_ref[:, sl] = d2c
            self32 = sel.astype(jnp.float32)
            g = g + _mm(self32, hi_ref[sl, :]) + _mm(self32, lo_ref[sl, :])
            mc = jnp.min(d2c, axis=1, keepdims=True)
            ac = jnp.min(jnp.where(d2c == mc, iota_c, n), axis=1,
                         keepdims=True)
            vals_n.append(mc)
            idxs_n.append(ac)
        rel = g[:, :3] - q
        gg = jnp.concatenate([rel, g[:, 3:wide]], axis=1)
        h = apply_mlp(gg)
        upd = jnp.logical_or(m <= r2, j == 0)
        acc = jnp.where(upd, jnp.maximum(acc, h), acc)
        return (acc, jnp.concatenate(vals_n, axis=1),
                jnp.concatenate(idxs_n, axis=1))

    acc0 = jnp.full((bq, out_dim), -inf, jnp.float32)
    acc, _, _ = jax.lax.fori_loop(0, nsample, step, (acc0, vals, idxs))
    return acc


def _down_body(nsample, r2, chunk, q_ref, sx_ref, hi_ref, lo_ref, w1_ref,
               b1_ref, w2_ref, b2_ref, o_ref, d2_ref):
    w1 = w1_ref[...]
    b1 = b1_ref[...]
    w2 = w2_ref[...]
    b2 = b2_ref[...]

    def mlp(gg):
        h = jnp.maximum(_mm(gg, w1) + b1, 0.0)
        return jnp.maximum(_mm(h, w2) + b2, 0.0)

    o_ref[0] = _select_pool(q_ref[0], sx_ref.at[0], hi_ref.at[0],
                            lo_ref.at[0], d2_ref, nsample, r2, chunk, mlp,
                            w2.shape[1])


def _up_body(nsample, r2, chunk, q_ref, sx_ref, hi_ref, lo_ref, fd_ref,
             w1_ref, b1_ref, w2_ref, b2_ref, o_ref, d2_ref):
    w1 = w1_ref[...]
    b1 = b1_ref[...]
    w2 = w2_ref[...]
    b2 = b2_ref[...]

    def mlp(gg):
        return jnp.maximum(_mm(gg, w1) + b1, 0.0)

    pooled = _select_pool(q_ref[0], sx_ref.at[0], hi_ref.at[0], lo_ref.at[0],
                          d2_ref, nsample, r2, chunk, mlp, w1.shape[1])
    hh = jnp.concatenate([pooled, fd_ref[0]], axis=1)
    o_ref[0] = jnp.maximum(_mm(hh, w2) + b2, 0.0)


def _stage(q_xyz, s_xyz, s_feat, w1, b1, w2, b2, nsample, radius, bq,
           q_feat=None):
    B, M, _ = q_xyz.shape
    _, N, C = s_feat.shape
    sall = jnp.concatenate([s_xyz, s_feat], axis=2)
    sxT = jnp.transpose(s_xyz, (0, 2, 1))
    hi = sall.astype(jnp.bfloat16).astype(jnp.float32)
    lo = sall - hi
    wide = C + 3
    chunk = min(512, N)
    F2 = w2.shape[1]
    up = q_feat is not None
    body = functools.partial(_up_body if up else _down_body,
                             nsample, radius * radius, chunk)
    in_specs = [
        pl.BlockSpec((1, bq, 3), lambda b, i: (b, i, 0)),
        pl.BlockSpec((1, 3, N), lambda b, i: (b, 0, 0)),
        pl.BlockSpec((1, N, wide), lambda b, i: (b, 0, 0)),
        pl.BlockSpec((1, N, wide), lambda b, i: (b, 0, 0)),
    ]
    args = [q_xyz, sxT, hi, lo]
    if up:
        in_specs.append(
            pl.BlockSpec((1, bq, q_feat.shape[2]), lambda b, i: (b, i, 0)))
        args.append(q_feat)
    in_specs += [
        pl.BlockSpec(w1.shape, lambda b, i: (0, 0)),
        pl.BlockSpec((1, w1.shape[1]), lambda b, i: (0, 0)),
        pl.BlockSpec(w2.shape, lambda b, i: (0, 0)),
        pl.BlockSpec((1, w2.shape[1]), lambda b, i: (0, 0)),
    ]
    args += [w1, b1.reshape(1, -1), w2, b2.reshape(1, -1)]
    return pl.pallas_call(
        body,
        grid=(B, M // bq),
        in_specs=in_specs,
        out_specs=pl.BlockSpec((1, bq, F2), lambda b, i: (b, i, 0)),
        out_shape=jax.ShapeDtypeStruct((B, M, F2), jnp.float32),
        scratch_shapes=[pltpu.VMEM((bq, N), jnp.float32)],
    )(*args)


def kernel(xyz, feat, d0_w0, d0_b0, d0_w1, d0_b1, d1_w0, d1_b0, d1_w1,
           d1_b1, d2_w0, d2_b0, d2_w1, d2_b1, u0_w0, u0_b0, u0_w1, u0_b1,
           u1_w0, u1_b0, u1_w1, u1_b1):
    B = xyz.shape[0]
    x1 = xyz[:, ::4]
    x2 = x1[:, ::4]
    x3 = x2[:, ::4]

    f1 = _stage(x1, xyz, feat, d0_w0, d0_b0, d0_w1, d0_b1,
                nsample=32, radius=0.1, bq=256)
    f2 = _stage(x2, x1, f1, d1_w0, d1_b0, d1_w1, d1_b1,
                nsample=32, radius=0.2, bq=512)
    f3 = _stage(x3, x2, f2, d2_w0, d2_b0, d2_w1, d2_b1,
                nsample=32, radius=0.4, bq=128)
    u0 = _stage(x2, x3, f3, u0_w0, u0_b0, u0_w1, u0_b1,
                nsample=32, radius=0.4, bq=512, q_feat=f2)
    u1 = _stage(x1, x2, u0, u1_w0, u1_b0, u1_w1, u1_b1,
                nsample=32, radius=0.2, bq=512, q_feat=f1)

    idx = jnp.broadcast_to(
        (jnp.arange(2048, dtype=jnp.int32) * 4)[None, :], (B, 2048))
    return (u1, x1, idx)


# speedup vs baseline: 9.3173x; 1.3739x over previous
import functools

import jax
import jax.numpy as jnp
from jax.experimental import pallas as pl
from jax.experimental.pallas import tpu as pltpu


def _mm(a, b):
    return jax.lax.dot_general(a, b, (((1,), (0,)), ((), ())),
                               precision=jax.lax.Precision.DEFAULT,
                               preferred_element_type=jnp.float32)


def _select_pool(q, sx_ref, ext_ref, d2_ref, wide, nsample, r2, chunk,
                 apply_mlp, out_dim):
    bq = q.shape[0]
    n = sx_ref.shape[1]
    nchunks = n // chunk
    p = ext_ref.shape[1] // 2
    inf = jnp.float32(jnp.inf)
    nf = jnp.float32(n)

    qq = jnp.sum(q * q, axis=1, keepdims=True)

    def iota_f(c):
        return (jax.lax.broadcasted_iota(jnp.int32, (bq, chunk), 1)
                + c * chunk).astype(jnp.float32)

    vals_l, idxs_l = [], []
    for c in range(nchunks):
        sl = pl.ds(c * chunk, chunk)
        sx = sx_ref[:, sl]
        ss = jnp.sum(sx * sx, axis=0, keepdims=True)
        qs = _mm(q, sx)
        d2c = (qq + ss) - 2.0 * qs
        d2_ref[:, sl] = d2c
        mc = jnp.min(d2c, axis=1, keepdims=True)
        ac = jnp.min(jnp.where(d2c == mc, iota_f(c), nf), axis=1,
                     keepdims=True)
        vals_l.append(mc)
        idxs_l.append(ac)
    vals = jnp.concatenate(vals_l, axis=1)
    idxs = jnp.concatenate(idxs_l, axis=1)

    def step(j, carry):
        acc, vals, idxs = carry
        m = jnp.min(vals, axis=1, keepdims=True)
        a = jnp.min(jnp.where(vals == m, idxs, nf), axis=1, keepdims=True)
        g2 = jnp.zeros((bq, 2 * p), jnp.float32)
        vals_n, idxs_n = [], []
        for c in range(nchunks):
            sl = pl.ds(c * chunk, chunk)
            d2c = d2_ref[:, sl]
            sel = iota_f(c) == a
            d2c = jnp.where(sel, inf, d2c)
            d2_ref[:, sl] = d2c
            g2 = g2 + _mm(sel.astype(jnp.float32), ext_ref[sl, :])
            mc = jnp.min(d2c, axis=1, keepdims=True)
            ac = jnp.min(jnp.where(d2c == mc, iota_f(c), nf), axis=1,
                         keepdims=True)
            vals_n.append(mc)
            idxs_n.append(ac)
        g = g2[:, :p] + g2[:, p:]
        rel = g[:, :3] - q
        gg = jnp.concatenate([rel, g[:, 3:wide]], axis=1)
        h = apply_mlp(gg)
        upd = jnp.logical_or(m <= r2, j == 0)
        acc = jnp.where(upd, jnp.maximum(acc, h), acc)
        return (acc, jnp.concatenate(vals_n, axis=1),
                jnp.concatenate(idxs_n, axis=1))

    acc0 = jnp.full((bq, out_dim), -inf, jnp.float32)
    acc, _, _ = jax.lax.fori_loop(0, nsample, step, (acc0, vals, idxs))
    return acc


def _down_body(nsample, r2, chunk, wide, q_ref, sx_ref, ext_ref, w1_ref,
               b1_ref, w2_ref, b2_ref, o_ref, d2_ref):
    w1 = w1_ref[...]
    b1 = b1_ref[...]
    w2 = w2_ref[...]
    b2 = b2_ref[...]

    def mlp(gg):
        h = jnp.maximum(_mm(gg, w1) + b1, 0.0)
        return jnp.maximum(_mm(h, w2) + b2, 0.0)

    o_ref[0] = _select_pool(q_ref[0], sx_ref.at[0], ext_ref.at[0], d2_ref,
                            wide, nsample, r2, chunk, mlp, w2.shape[1])


def _up_body(nsample, r2, chunk, wide, q_ref, sx_ref, ext_ref, fd_ref,
             w1_ref, b1_ref, w2_ref, b2_ref, o_ref, d2_ref):
    w1 = w1_ref[...]
    b1 = b1_ref[...]
    w2 = w2_ref[...]
    b2 = b2_ref[...]

    def mlp(gg):
        return jnp.maximum(_mm(gg, w1) + b1, 0.0)

    pooled = _select_pool(q_ref[0], sx_ref.at[0], ext_ref.at[0], d2_ref,
                          wide, nsample, r2, chunk, mlp, w1.shape[1])
    hh = jnp.concatenate([pooled, fd_ref[0]], axis=1)
    o_ref[0] = jnp.maximum(_mm(hh, w2) + b2, 0.0)


def _stage(q_xyz, s_xyz, s_feat, w1, b1, w2, b2, nsample, radius, bq,
           q_feat=None):
    B, M, _ = q_xyz.shape
    _, N, C = s_feat.shape
    sall = jnp.concatenate([s_xyz, s_feat], axis=2)
    sxT = jnp.transpose(s_xyz, (0, 2, 1))
    hi = sall.astype(jnp.bfloat16).astype(jnp.float32)
    lo = sall - hi
    wide = C + 3
    p = 128 * ((wide + 127) // 128)
    pad = jnp.zeros((B, N, p - wide), jnp.float32)
    ext = jnp.concatenate([hi, pad, lo, pad], axis=2)
    chunk = min(1024, N)
    F2 = w2.shape[1]
    up = q_feat is not None
    body = functools.partial(_up_body if up else _down_body,
                             nsample, radius * radius, chunk, wide)
    in_specs = [
        pl.BlockSpec((1, bq, 3), lambda b, i: (b, i, 0)),
        pl.BlockSpec((1, 3, N), lambda b, i: (b, 0, 0)),
        pl.BlockSpec((1, N, 2 * p), lambda b, i: (b, 0, 0)),
    ]
    args = [q_xyz, sxT, ext]
    if up:
        in_specs.append(
            pl.BlockSpec((1, bq, q_feat.shape[2]), lambda b, i: (b, i, 0)))
        args.append(q_feat)
    in_specs += [
        pl.BlockSpec(w1.shape, lambda b, i: (0, 0)),
        pl.BlockSpec((1, w1.shape[1]), lambda b, i: (0, 0)),
        pl.BlockSpec(w2.shape, lambda b, i: (0, 0)),
        pl.BlockSpec((1, w2.shape[1]), lambda b, i: (0, 0)),
    ]
    args += [w1, b1.reshape(1, -1), w2, b2.reshape(1, -1)]
    return pl.pallas_call(
        body,
        grid=(B, M // bq),
        in_specs=in_specs,
        out_specs=pl.BlockSpec((1, bq, F2), lambda b, i: (b, i, 0)),
        out_shape=jax.ShapeDtypeStruct((B, M, F2), jnp.float32),
        scratch_shapes=[pltpu.VMEM((bq, N), jnp.float32)],
    )(*args)


def kernel(xyz, feat, d0_w0, d0_b0, d0_w1, d0_b1, d1_w0, d1_b0, d1_w1,
           d1_b1, d2_w0, d2_b0, d2_w1, d2_b1, u0_w0, u0_b0, u0_w1, u0_b1,
           u1_w0, u1_b0, u1_w1, u1_b1):
    B = xyz.shape[0]
    x1 = xyz[:, ::4]
    x2 = x1[:, ::4]
    x3 = x2[:, ::4]

    f1 = _stage(x1, xyz, feat, d0_w0, d0_b0, d0_w1, d0_b1,
                nsample=32, radius=0.1, bq=256)
    f2 = _stage(x2, x1, f1, d1_w0, d1_b0, d1_w1, d1_b1,
                nsample=32, radius=0.2, bq=512)
    f3 = _stage(x3, x2, f2, d2_w0, d2_b0, d2_w1, d2_b1,
                nsample=32, radius=0.4, bq=128)
    u0 = _stage(x2, x3, f3, u0_w0, u0_b0, u0_w1, u0_b1,
                nsample=32, radius=0.4, bq=512, q_feat=f2)
    u1 = _stage(x1, x2, u0, u1_w0, u1_b0, u1_w1, u1_b1,
                nsample=32, radius=0.2, bq=512, q_feat=f1)

    idx = jnp.broadcast_to(
        (jnp.arange(2048, dtype=jnp.int32) * 4)[None, :], (B, 2048))
    return (u1, x1, idx)


# speedup vs baseline: 9.8962x; 1.0621x over previous
import functools

import jax
import jax.numpy as jnp
from jax.experimental import pallas as pl
from jax.experimental.pallas import tpu as pltpu


def _mm(a, b):
    return jax.lax.dot_general(a, b, (((1,), (0,)), ((), ())),
                               precision=jax.lax.Precision.DEFAULT,
                               preferred_element_type=jnp.float32)


def _select_pool(q, sx_ref, ext_ref, d2_ref, wide, nsample, r2, chunk,
                 apply_mlp, out_dim):
    bq = q.shape[0]
    n = sx_ref.shape[1]
    nchunks = n // chunk
    p = ext_ref.shape[1] // 2
    inf = jnp.float32(jnp.inf)
    nf = jnp.float32(n)

    qq = jnp.sum(q * q, axis=1, keepdims=True)

    def iota_f(c):
        return (jax.lax.broadcasted_iota(jnp.int32, (1, chunk), 1)
                + c * chunk).astype(jnp.float32)

    vals_l, idxs_l = [], []
    for c in range(nchunks):
        sl = pl.ds(c * chunk, chunk)
        sx = sx_ref[:, sl]
        ss = jnp.sum(sx * sx, axis=0, keepdims=True)
        qs = _mm(q, sx)
        d2c = (qq + ss) - 2.0 * qs
        d2_ref[:, sl] = d2c
        mc = jnp.min(d2c, axis=1, keepdims=True)
        ac = jnp.min(jnp.where(d2c == mc, iota_f(c), nf), axis=1,
                     keepdims=True)
        vals_l.append(mc)
        idxs_l.append(ac)
    vals = jnp.concatenate(vals_l, axis=1)
    idxs = jnp.concatenate(idxs_l, axis=1)

    def step(j, carry):
        acc, vals, idxs = carry
        m = jnp.min(vals, axis=1, keepdims=True)
        a = jnp.min(jnp.where(vals == m, idxs, nf), axis=1, keepdims=True)
        g2 = jnp.zeros((bq, 2 * p), jnp.float32)
        vals_n, idxs_n = [], []
        for c in range(nchunks):
            sl = pl.ds(c * chunk, chunk)
            d2c = d2_ref[:, sl]
            sel = iota_f(c) == a
            d2c = jnp.where(sel, inf, d2c)
            d2_ref[:, sl] = d2c
            g2 = g2 + _mm(sel.astype(jnp.float32), ext_ref[sl, :])
            mc = jnp.min(d2c, axis=1, keepdims=True)
            ac = jnp.min(jnp.where(d2c == mc, iota_f(c), nf), axis=1,
                         keepdims=True)
            vals_n.append(mc)
            idxs_n.append(ac)
        g = g2[:, :p] + g2[:, p:]
        rel = g[:, :3] - q
        gg = jnp.concatenate([rel, g[:, 3:wide]], axis=1)
        h = apply_mlp(gg)
        upd = jnp.logical_or(m <= r2, j == 0)
        acc = jnp.where(upd, jnp.maximum(acc, h), acc)
        return (acc, jnp.concatenate(vals_n, axis=1),
                jnp.concatenate(idxs_n, axis=1))

    acc0 = jnp.full((bq, out_dim), -inf, jnp.float32)
    acc, _, _ = jax.lax.fori_loop(0, nsample, step, (acc0, vals, idxs))
    return acc


def _down_body(nsample, r2, chunk, wide, q_ref, sx_ref, ext_ref, w1_ref,
               b1_ref, w2_ref, b2_ref, o_ref, d2_ref):
    w1 = w1_ref[...]
    b1 = b1_ref[...]
    w2 = w2_ref[...]
    b2 = b2_ref[...]

    def mlp(gg):
        h = jnp.maximum(_mm(gg, w1) + b1, 0.0)
        return jnp.maximum(_mm(h, w2) + b2, 0.0)

    o_ref[0] = _select_pool(q_ref[0], sx_ref.at[0], ext_ref.at[0], d2_ref,
                            wide, nsample, r2, chunk, mlp, w2.shape[1])


def _up_body(nsample, r2, chunk, wide, q_ref, sx_ref, ext_ref, fd_ref,
             w1_ref, b1_ref, w2_ref, b2_ref, o_ref, d2_ref):
    w1 = w1_ref[...]
    b1 = b1_ref[...]
    w2 = w2_ref[...]
    b2 = b2_ref[...]

    def mlp(gg):
        return jnp.maximum(_mm(gg, w1) + b1, 0.0)

    pooled = _select_pool(q_ref[0], sx_ref.at[0], ext_ref.at[0], d2_ref,
                          wide, nsample, r2, chunk, mlp, w1.shape[1])
    hh = jnp.concatenate([pooled, fd_ref[0]], axis=1)
    o_ref[0] = jnp.maximum(_mm(hh, w2) + b2, 0.0)


def _stage(q_xyz, s_xyz, s_feat, w1, b1, w2, b2, nsample, radius, bq,
           q_feat=None):
    B, M, _ = q_xyz.shape
    _, N, C = s_feat.shape
    sall = jnp.concatenate([s_xyz, s_feat], axis=2)
    sxT = jnp.transpose(s_xyz, (0, 2, 1))
    hi = sall.astype(jnp.bfloat16).astype(jnp.float32)
    lo = sall - hi
    wide = C + 3
    p = 128 * ((wide + 127) // 128)
    pad = jnp.zeros((B, N, p - wide), jnp.float32)
    ext = jnp.concatenate([hi, pad, lo, pad], axis=2)
    chunk = min(1024, N)
    F2 = w2.shape[1]
    up = q_feat is not None
    body = functools.partial(_up_body if up else _down_body,
                             nsample, radius * radius, chunk, wide)
    in_specs = [
        pl.BlockSpec((1, bq, 3), lambda b, i: (b, i, 0)),
        pl.BlockSpec((1, 3, N), lambda b, i: (b, 0, 0)),
        pl.BlockSpec((1, N, 2 * p), lambda b, i: (b, 0, 0)),
    ]
    args = [q_xyz, sxT, ext]
    if up:
        in_specs.append(
            pl.BlockSpec((1, bq, q_feat.shape[2]), lambda b, i: (b, i, 0)))
        args.append(q_feat)
    in_specs += [
        pl.BlockSpec(w1.shape, lambda b, i: (0, 0)),
        pl.BlockSpec((1, w1.shape[1]), lambda b, i: (0, 0)),
        pl.BlockSpec(w2.shape, lambda b, i: (0, 0)),
        pl.BlockSpec((1, w2.shape[1]), lambda b, i: (0, 0)),
    ]
    args += [w1, b1.reshape(1, -1), w2, b2.reshape(1, -1)]
    return pl.pallas_call(
        body,
        grid=(B, M // bq),
        in_specs=in_specs,
        out_specs=pl.BlockSpec((1, bq, F2), lambda b, i: (b, i, 0)),
        out_shape=jax.ShapeDtypeStruct((B, M, F2), jnp.float32),
        scratch_shapes=[pltpu.VMEM((bq, N), jnp.float32)],
    )(*args)


def kernel(xyz, feat, d0_w0, d0_b0, d0_w1, d0_b1, d1_w0, d1_b0, d1_w1,
           d1_b1, d2_w0, d2_b0, d2_w1, d2_b1, u0_w0, u0_b0, u0_w1, u0_b1,
           u1_w0, u1_b0, u1_w1, u1_b1):
    B = xyz.shape[0]
    x1 = xyz[:, ::4]
    x2 = x1[:, ::4]
    x3 = x2[:, ::4]

    f1 = _stage(x1, xyz, feat, d0_w0, d0_b0, d0_w1, d0_b1,
                nsample=32, radius=0.1, bq=512)
    f2 = _stage(x2, x1, f1, d1_w0, d1_b0, d1_w1, d1_b1,
                nsample=32, radius=0.2, bq=512)
    f3 = _stage(x3, x2, f2, d2_w0, d2_b0, d2_w1, d2_b1,
                nsample=32, radius=0.4, bq=128)
    u0 = _stage(x2, x3, f3, u0_w0, u0_b0, u0_w1, u0_b1,
                nsample=32, radius=0.4, bq=512, q_feat=f2)
    u1 = _stage(x1, x2, u0, u1_w0, u1_b0, u1_w1, u1_b1,
                nsample=32, radius=0.2, bq=512, q_feat=f1)

    idx = jnp.broadcast_to(
        (jnp.arange(2048, dtype=jnp.int32) * 4)[None, :], (B, 2048))
    return (u1, x1, idx)


# speedup vs baseline: 10.5282x; 1.0639x over previous
import functools

import jax
import jax.numpy as jnp
from jax.experimental import pallas as pl
from jax.experimental.pallas import tpu as pltpu


def _mm(a, b):
    return jax.lax.dot_general(a, b, (((1,), (0,)), ((), ())),
                               precision=jax.lax.Precision.DEFAULT,
                               preferred_element_type=jnp.float32)


def _select_pool(q, sx_ref, ext_ref, d2_ref, g_ref, ms_ref, wide, nsample,
                 r2, chunk, apply_mlp, out_dim):
    bq = q.shape[0]
    n = sx_ref.shape[1]
    nchunks = n // chunk
    inf = jnp.float32(jnp.inf)
    nf = jnp.float32(n)

    qq = jnp.sum(q * q, axis=1, keepdims=True)

    def iota_f(c):
        return (jax.lax.broadcasted_iota(jnp.int32, (1, chunk), 1)
                + c * chunk).astype(jnp.float32)

    vals_l, idxs_l = [], []
    for c in range(nchunks):
        sl = pl.ds(c * chunk, chunk)
        sx = sx_ref[:, sl]
        ss = jnp.sum(sx * sx, axis=0, keepdims=True)
        qs = _mm(q, sx)
        d2c = (qq + ss) - 2.0 * qs
        d2_ref[:, sl] = d2c
        mc = jnp.min(d2c, axis=1, keepdims=True)
        ac = jnp.min(jnp.where(d2c == mc, iota_f(c), nf), axis=1,
                     keepdims=True)
        vals_l.append(mc)
        idxs_l.append(ac)
    vals = jnp.concatenate(vals_l, axis=1)
    idxs = jnp.concatenate(idxs_l, axis=1)

    def step(j, carry):
        vals, idxs = carry
        m = jnp.min(vals, axis=1, keepdims=True)
        a = jnp.min(jnp.where(vals == m, idxs, nf), axis=1, keepdims=True)
        g2 = jnp.zeros((bq, 2 * wide), jnp.float32)
        vals_n, idxs_n = [], []
        for c in range(nchunks):
            sl = pl.ds(c * chunk, chunk)
            d2c = d2_ref[:, sl]
            sel = iota_f(c) == a
            d2c = jnp.where(sel, inf, d2c)
            d2_ref[:, sl] = d2c
            g2 = g2 + _mm(sel.astype(jnp.float32), ext_ref[sl, :])
            mc = jnp.min(d2c, axis=1, keepdims=True)
            ac = jnp.min(jnp.where(d2c == mc, iota_f(c), nf), axis=1,
                         keepdims=True)
            vals_n.append(mc)
            idxs_n.append(ac)
        g_ref[j] = g2[:, :wide] + g2[:, wide:]
        ms_ref[j] = m
        return (jnp.concatenate(vals_n, axis=1),
                jnp.concatenate(idxs_n, axis=1))

    jax.lax.fori_loop(0, nsample, step, (vals, idxs))

    g_all = g_ref[...]
    rel = g_all[:, :, :3] - q[None]
    gg = jnp.concatenate([rel, g_all[:, :, 3:]], axis=2)
    h = apply_mlp(gg.reshape(nsample * bq, wide))
    h = h.reshape(nsample, bq, out_dim)
    first = jax.lax.broadcasted_iota(jnp.int32, (nsample, 1, 1), 0) == 0
    upd = jnp.logical_or(ms_ref[...] <= r2, first)
    return jnp.max(jnp.where(upd, h, -inf), axis=0)


def _down_body(nsample, r2, chunk, wide, q_ref, sx_ref, ext_ref, w1_ref,
               b1_ref, w2_ref, b2_ref, o_ref, d2_ref, g_ref, ms_ref):
    w1 = w1_ref[...]
    b1 = b1_ref[...]
    w2 = w2_ref[...]
    b2 = b2_ref[...]

    def mlp(gg):
        h = jnp.maximum(_mm(gg, w1) + b1, 0.0)
        return jnp.maximum(_mm(h, w2) + b2, 0.0)

    o_ref[0] = _select_pool(q_ref[0], sx_ref.at[0], ext_ref.at[0], d2_ref,
                            g_ref, ms_ref, wide, nsample, r2, chunk, mlp,
                            w2.shape[1])


def _up_body(nsample, r2, chunk, wide, q_ref, sx_ref, ext_ref, fd_ref,
             w1_ref, b1_ref, w2_ref, b2_ref, o_ref, d2_ref, g_ref, ms_ref):
    w1 = w1_ref[...]
    b1 = b1_ref[...]
    w2 = w2_ref[...]
    b2 = b2_ref[...]

    def mlp(gg):
        return jnp.maximum(_mm(gg, w1) + b1, 0.0)

    pooled = _select_pool(q_ref[0], sx_ref.at[0], ext_ref.at[0], d2_ref,
                          g_ref, ms_ref, wide, nsample, r2, chunk, mlp,
                          w1.shape[1])
    hh = jnp.concatenate([pooled, fd_ref[0]], axis=1)
    o_ref[0] = jnp.maximum(_mm(hh, w2) + b2, 0.0)


def _stage(q_xyz, s_xyz, s_feat, w1, b1, w2, b2, nsample, radius, bq,
           q_feat=None):
    B, M, _ = q_xyz.shape
    _, N, C = s_feat.shape
    sall = jnp.concatenate([s_xyz, s_feat], axis=2)
    sxT = jnp.transpose(s_xyz, (0, 2, 1))
    hi = sall.astype(jnp.bfloat16).astype(jnp.float32)
    lo = sall - hi
    wide = C + 3
    ext = jnp.concatenate([hi, lo], axis=2)
    chunk = min(1024, N)
    F2 = w2.shape[1]
    up = q_feat is not None
    body = functools.partial(_up_body if up else _down_body,
                             nsample, radius * radius, chunk, wide)
    in_specs = [
        pl.BlockSpec((1, bq, 3), lambda b, i: (b, i, 0)),
        pl.BlockSpec((1, 3, N), lambda b, i: (b, 0, 0)),
        pl.BlockSpec((1, N, 2 * wide), lambda b, i: (b, 0, 0)),
    ]
    args = [q_xyz, sxT, ext]
    if up:
        in_specs.append(
            pl.BlockSpec((1, bq, q_feat.shape[2]), lambda b, i: (b, i, 0)))
        args.append(q_feat)
    in_specs += [
        pl.BlockSpec(w1.shape, lambda b, i: (0, 0)),
        pl.BlockSpec((1, w1.shape[1]), lambda b, i: (0, 0)),
        pl.BlockSpec(w2.shape, lambda b, i: (0, 0)),
        pl.BlockSpec((1, w2.shape[1]), lambda b, i: (0, 0)),
    ]
    args += [w1, b1.reshape(1, -1), w2, b2.reshape(1, -1)]
    return pl.pallas_call(
        body,
        grid=(B, M // bq),
        in_specs=in_specs,
        out_specs=pl.BlockSpec((1, bq, F2), lambda b, i: (b, i, 0)),
        out_shape=jax.ShapeDtypeStruct((B, M, F2), jnp.float32),
        scratch_shapes=[pltpu.VMEM((bq, N), jnp.float32),
                        pltpu.VMEM((nsample, bq, wide), jnp.float32),
                        pltpu.VMEM((nsample, bq, 1), jnp.float32)],
    )(*args)


def kernel(xyz, feat, d0_w0, d0_b0, d0_w1, d0_b1, d1_w0, d1_b0, d1_w1,
           d1_b1, d2_w0, d2_b0, d2_w1, d2_b1, u0_w0, u0_b0, u0_w1, u0_b1,
           u1_w0, u1_b0, u1_w1, u1_b1):
    B = xyz.shape[0]
    x1 = xyz[:, ::4]
    x2 = x1[:, ::4]
    x3 = x2[:, ::4]

    f1 = _stage(x1, xyz, feat, d0_w0, d0_b0, d0_w1, d0_b1,
                nsample=32, radius=0.1, bq=512)
    f2 = _stage(x2, x1, f1, d1_w0, d1_b0, d1_w1, d1_b1,
                nsample=32, radius=0.2, bq=512)
    f3 = _stage(x3, x2, f2, d2_w0, d2_b0, d2_w1, d2_b1,
                nsample=32, radius=0.4, bq=128)
    u0 = _stage(x2, x3, f3, u0_w0, u0_b0, u0_w1, u0_b1,
                nsample=32, radius=0.4, bq=512, q_feat=f2)
    u1 = _stage(x1, x2, u0, u1_w0, u1_b0, u1_w1, u1_b1,
                nsample=32, radius=0.2, bq=512, q_feat=f1)

    idx = jnp.broadcast_to(
        (jnp.arange(2048, dtype=jnp.int32) * 4)[None, :], (B, 2048))
    return (u1, x1, idx)
